# Initial kernel scaffold; baseline (speedup 1.0000x reference)
import jax
import jax.numpy as jnp
from jax.experimental import pallas as pl


def kernel(x, emb):
    raise NotImplementedError("write your pallas kernel here")


# speedup vs baseline: 4.8091x; 4.8091x over previous
import functools

import jax
import jax.numpy as jnp
from jax import lax
from jax.experimental import pallas as pl
from jax.experimental.pallas import tpu as pltpu
from jax.experimental.pallas import tpu_sc as plsc

_LANE = 128
_K = 8
_NW = 32


def _build_gather(n_rows, d, v):
    rows_per_w = n_rows // _NW
    n_steps = rows_per_w // _K

    mesh = plsc.VectorSubcoreMesh(core_axis_name="c", subcore_axis_name="s")

    @functools.partial(
        pl.kernel,
        mesh=mesh,
        out_type=jax.ShapeDtypeStruct((n_rows, _LANE, d), jnp.float32),
        compiler_params=pltpu.CompilerParams(use_tc_tiling_on_sc=False),
        scratch_types=[
            pltpu.VMEM((_K, _LANE), jnp.int32),
            pltpu.VMEM((_K, _LANE, d), jnp.float32),
            pltpu.SemaphoreType.DMA,
        ],
    )
    def gather(idx_hbm, table_hbm, out_hbm, idx_v, rows_v, sem):
        wid = lax.axis_index("s") * 2 + lax.axis_index("c")
        base = wid * rows_per_w

        def step(g, carry):
            row0 = base + g * _K
            pltpu.sync_copy(idx_hbm.at[pl.ds(row0, _K)], idx_v)
            copies = []
            for j in range(_K):
                copies.append(
                    pltpu.async_copy(
                        table_hbm.at[idx_v.at[j]], rows_v.at[j], sem))
            for c in copies:
                c.wait()
            pltpu.sync_copy(rows_v, out_hbm.at[pl.ds(row0, _K)])
            return carry

        lax.fori_loop(0, n_steps, step, 0, unroll=False)

    return gather


def kernel(x, emb):
    b0, b1 = x.shape
    v, d = emb.shape
    n = b0 * b1
    n_rows = n // _LANE
    idx = x.reshape(n_rows, _LANE).astype(jnp.int32)
    out = _build_gather(n_rows, d, v)(idx, emb)
    return out.reshape(b0, b1, d)


# speedup vs baseline: 5.0492x; 1.0499x over previous
import functools

import jax
import jax.numpy as jnp
from jax import lax
from jax.experimental import pallas as pl
from jax.experimental.pallas import tpu as pltpu
from jax.experimental.pallas import tpu_sc as plsc

_LANE = 128
_K = 8
_NBUF = 2
_NW = 32


def _build_gather(n_rows, d, v):
    rows_per_w = n_rows // _NW
    n_steps = rows_per_w // _K
    n_rounds = n_steps // _NBUF

    mesh = plsc.VectorSubcoreMesh(core_axis_name="c", subcore_axis_name="s")

    @functools.partial(
        pl.kernel,
        mesh=mesh,
        out_type=jax.ShapeDtypeStruct((n_rows, _LANE, d), jnp.float32),
        compiler_params=pltpu.CompilerParams(use_tc_tiling_on_sc=False),
        scratch_types=[
            pltpu.VMEM((_NBUF, _K, _LANE), jnp.int32),
            pltpu.VMEM((_NBUF, _K, _LANE, d), jnp.float32),
            pltpu.SemaphoreType.DMA((_NBUF,)),
            pltpu.SemaphoreType.DMA((_NBUF,)),
            pltpu.SemaphoreType.DMA((_NBUF,)),
        ],
    )
    def gather(idx_hbm, table_hbm, out_hbm, idx_v, rows_v, sem_i, sem_g,
               sem_s):
        wid = lax.axis_index("s") * 2 + lax.axis_index("c")
        base = wid * rows_per_w

        def fire_gathers(b):
            for j in range(_K):
                pltpu.async_copy(
                    table_hbm.at[idx_v.at[b, j]], rows_v.at[b, j],
                    sem_g.at[b])

        def drain_gathers(b):
            pltpu.make_async_copy(
                out_hbm.at[pl.ds(0, _K)], rows_v.at[b], sem_g.at[b]).wait()

        for b in range(_NBUF):
            pltpu.sync_copy(idx_hbm.at[pl.ds(base + b * _K, _K)],
                            idx_v.at[b])
            fire_gathers(b)

        def round_body(r, carry):
            for b in range(_NBUF):
                t = r * _NBUF + b
                row0 = base + t * _K
                drain_gathers(b)
                pltpu.async_copy(rows_v.at[b], out_hbm.at[pl.ds(row0, _K)],
                                 sem_s.at[b])

                @pl.when(r < n_rounds - 1)
                def _prepare():
                    nxt = base + (t + _NBUF) * _K
                    pltpu.async_copy(idx_hbm.at[pl.ds(nxt, _K)],
                                     idx_v.at[b], sem_i.at[b])
                    pltpu.make_async_copy(
                        rows_v.at[b], out_hbm.at[pl.ds(0, _K)],
                        sem_s.at[b]).wait()
                    pltpu.make_async_copy(
                        idx_hbm.at[pl.ds(0, _K)], idx_v.at[b],
                        sem_i.at[b]).wait()
                    fire_gathers(b)
            return carry

        lax.fori_loop(0, n_rounds, round_body, 0, unroll=False)

        for b in range(_NBUF):
            pltpu.make_async_copy(
                rows_v.at[b], out_hbm.at[pl.ds(0, _K)], sem_s.at[b]).wait()

    return gather


def kernel(x, emb):
    b0, b1 = x.shape
    v, d = emb.shape
    n = b0 * b1
    n_rows = n // _LANE
    idx = x.reshape(n_rows, _LANE).astype(jnp.int32)
    out = _build_gather(n_rows, d, v)(idx, emb)
    return out.reshape(b0, b1, d)
